# Initial kernel scaffold; baseline (speedup 1.0000x reference)
import jax
import jax.numpy as jnp
from jax.experimental import pallas as pl


def kernel(inputs, embedding_table):
    raise NotImplementedError("write your pallas kernel here")


# speedup vs baseline: 4.8043x; 4.8043x over previous
import functools

import jax
import jax.numpy as jnp
from jax import lax
from jax.experimental import pallas as pl
from jax.experimental.pallas import tpu as pltpu
from jax.experimental.pallas import tpu_sc as plsc

VOCAB = 100000
EMBED = 128
BATCH = 1024
SEQ = 200

_info = plsc.get_sparse_core_info()
_NC, _NS = _info.num_cores, _info.num_subcores
_NW = _NC * _NS
_TOTAL = BATCH * SEQ
_PER_W = _TOTAL // _NW
_CL = 128
_NCHUNK = _PER_W // _CL


@functools.partial(
    pl.kernel,
    mesh=plsc.VectorSubcoreMesh(core_axis_name="c", subcore_axis_name="s"),
    out_type=jax.ShapeDtypeStruct((_TOTAL, EMBED), jnp.float32),
    scratch_types=[
        pltpu.VMEM((_NCHUNK, _CL), jnp.int32),
        pltpu.VMEM((_CL, EMBED), jnp.float32),
        pltpu.SemaphoreType.DMA,
    ],
)
def _gather_kernel(idx_hbm, table_hbm, out_hbm, idx_v, rows_v, sem):
    wid = lax.axis_index("s") * _NC + lax.axis_index("c")
    base = wid * _PER_W
    pltpu.sync_copy(idx_hbm.at[wid], idx_v)

    def body(j, _):
        pltpu.async_copy(table_hbm.at[idx_v.at[j]], rows_v, sem).wait()
        pltpu.sync_copy(rows_v, out_hbm.at[pl.ds(base + j * _CL, _CL)])
        return 0

    lax.fori_loop(0, _NCHUNK, body, 0)


def kernel(inputs, embedding_table):
    idx = jnp.reshape(inputs.astype(jnp.int32), (_NW, _NCHUNK, _CL))
    out = _gather_kernel(idx, embedding_table)
    return (jnp.reshape(out, (BATCH, SEQ, EMBED)), embedding_table)


# speedup vs baseline: 6.1327x; 1.2765x over previous
import functools

import jax
import jax.numpy as jnp
from jax import lax
from jax.experimental import pallas as pl
from jax.experimental.pallas import tpu as pltpu
from jax.experimental.pallas import tpu_sc as plsc

VOCAB = 100000
EMBED = 128
BATCH = 1024
SEQ = 200

_info = plsc.get_sparse_core_info()
_NC, _NS = _info.num_cores, _info.num_subcores
_NW = _NC * _NS
_TOTAL = BATCH * SEQ
_PER_W = _TOTAL // _NW
_CL = 128
_NCHUNK = _PER_W // _CL


_NBUF = 2


@functools.partial(
    pl.kernel,
    mesh=plsc.VectorSubcoreMesh(core_axis_name="c", subcore_axis_name="s"),
    out_type=jax.ShapeDtypeStruct((_TOTAL, EMBED), jnp.float32),
    scratch_types=[
        pltpu.VMEM((_NCHUNK, _CL), jnp.int32),
        pltpu.VMEM((_CL, EMBED), jnp.float32),
        pltpu.VMEM((_CL, EMBED), jnp.float32),
        pltpu.SemaphoreType.DMA,
        pltpu.SemaphoreType.DMA,
        pltpu.SemaphoreType.DMA,
        pltpu.SemaphoreType.DMA,
    ],
)
def _gather_kernel(idx_hbm, table_hbm, out_hbm, idx_v,
                   rows0, rows1, gsem0, gsem1, osem0, osem1):
    wid = lax.axis_index("s") * _NC + lax.axis_index("c")
    base = wid * _PER_W
    bufs = ((rows0, gsem0, osem0), (rows1, gsem1, osem1))
    pltpu.sync_copy(idx_hbm.at[wid], idx_v)

    for b in range(_NBUF):
        rows, gsem, _ = bufs[b]
        pltpu.async_copy(table_hbm.at[idx_v.at[b]], rows, gsem)

    def body(g, _):
        for b in range(_NBUF):
            j = g * _NBUF + b
            rows, gsem, osem = bufs[b]
            dst = out_hbm.at[pl.ds(base + j * _CL, _CL)]
            pltpu.make_async_copy(table_hbm.at[idx_v.at[j]], rows, gsem).wait()
            pltpu.async_copy(rows, dst, osem)

            @pl.when(j + _NBUF < _NCHUNK)
            def _():
                pltpu.make_async_copy(rows, dst, osem).wait()
                pltpu.async_copy(table_hbm.at[idx_v.at[j + _NBUF]], rows, gsem)

        return 0

    lax.fori_loop(0, _NCHUNK // _NBUF, body, 0)

    for b in range(_NBUF):
        j = _NCHUNK - _NBUF + b
        rows, _, osem = bufs[b]
        dst = out_hbm.at[pl.ds(base + j * _CL, _CL)]
        pltpu.make_async_copy(rows, dst, osem).wait()


def kernel(inputs, embedding_table):
    idx = jnp.reshape(inputs.astype(jnp.int32), (_NW, _NCHUNK, _CL))
    out = _gather_kernel(idx, embedding_table)
    return (jnp.reshape(out, (BATCH, SEQ, EMBED)), embedding_table)


# speedup vs baseline: 6.1847x; 1.0085x over previous
import functools

import jax
import jax.numpy as jnp
from jax import lax
from jax.experimental import pallas as pl
from jax.experimental.pallas import tpu as pltpu
from jax.experimental.pallas import tpu_sc as plsc

VOCAB = 100000
EMBED = 128
BATCH = 1024
SEQ = 200

_info = plsc.get_sparse_core_info()
_NC, _NS = _info.num_cores, _info.num_subcores
_NW = _NC * _NS
_TOTAL = BATCH * SEQ
_PER_W = _TOTAL // _NW
_CL = 64
_NCHUNK = _PER_W // _CL
_NBUF = 4


@functools.partial(
    pl.kernel,
    mesh=plsc.VectorSubcoreMesh(core_axis_name="c", subcore_axis_name="s"),
    out_type=jax.ShapeDtypeStruct((_TOTAL, EMBED), jnp.float32),
    scratch_types=(
        [pltpu.VMEM((_NCHUNK, _CL), jnp.int32)]
        + [pltpu.VMEM((_CL, EMBED), jnp.float32)] * _NBUF
        + [pltpu.SemaphoreType.DMA] * (2 * _NBUF)
    ),
)
def _gather_kernel(idx_hbm, table_hbm, out_hbm, idx_v, *bufs_and_sems):
    rows_bufs = bufs_and_sems[:_NBUF]
    gsems = bufs_and_sems[_NBUF:2 * _NBUF]
    osems = bufs_and_sems[2 * _NBUF:]
    wid = lax.axis_index("s") * _NC + lax.axis_index("c")
    base = wid * _PER_W
    bufs = tuple(zip(rows_bufs, gsems, osems))
    pltpu.sync_copy(idx_hbm.at[wid], idx_v)

    for b in range(_NBUF):
        rows, gsem, _ = bufs[b]
        pltpu.async_copy(table_hbm.at[idx_v.at[b]], rows, gsem)

    def body(g, _):
        for b in range(_NBUF):
            j = g * _NBUF + b
            rows, gsem, osem = bufs[b]
            dst = out_hbm.at[pl.ds(base + j * _CL, _CL)]
            pltpu.make_async_copy(table_hbm.at[idx_v.at[j]], rows, gsem).wait()
            pltpu.async_copy(rows, dst, osem)

            @pl.when(j + _NBUF < _NCHUNK)
            def _():
                pltpu.make_async_copy(rows, dst, osem).wait()
                pltpu.async_copy(table_hbm.at[idx_v.at[j + _NBUF]], rows, gsem)

        return 0

    lax.fori_loop(0, _NCHUNK // _NBUF, body, 0)

    for b in range(_NBUF):
        j = _NCHUNK - _NBUF + b
        rows, _, osem = bufs[b]
        dst = out_hbm.at[pl.ds(base + j * _CL, _CL)]
        pltpu.make_async_copy(rows, dst, osem).wait()


def kernel(inputs, embedding_table):
    idx = jnp.reshape(inputs.astype(jnp.int32), (_NW, _NCHUNK, _CL))
    out = _gather_kernel(idx, embedding_table)
    return (jnp.reshape(out, (BATCH, SEQ, EMBED)), embedding_table)


# speedup vs baseline: 6.2376x; 1.0085x over previous
import functools

import jax
import jax.numpy as jnp
from jax import lax
from jax.experimental import pallas as pl
from jax.experimental.pallas import tpu as pltpu
from jax.experimental.pallas import tpu_sc as plsc

VOCAB = 100000
EMBED = 128
BATCH = 1024
SEQ = 200

_info = plsc.get_sparse_core_info()
_NC, _NS = _info.num_cores, _info.num_subcores
_NW = _NC * _NS
_BPW = BATCH // _NW
_HALF = SEQ // 2
_NBUF = 4


@functools.partial(
    pl.kernel,
    mesh=plsc.VectorSubcoreMesh(core_axis_name="c", subcore_axis_name="s"),
    out_type=jax.ShapeDtypeStruct((BATCH, SEQ, EMBED), jnp.float32),
    scratch_types=(
        [pltpu.VMEM((2 * _BPW, _HALF), jnp.int32)]
        + [pltpu.VMEM((SEQ, EMBED), jnp.float32)] * _NBUF
        + [pltpu.SemaphoreType.DMA] * (2 * _NBUF)
    ),
)
def _gather_kernel(idx_hbm, table_hbm, out_hbm, idx_v, *bufs_and_sems):
    rows_bufs = bufs_and_sems[:_NBUF]
    gsems = bufs_and_sems[_NBUF:2 * _NBUF]
    osems = bufs_and_sems[2 * _NBUF:]
    bufs = tuple(zip(rows_bufs, gsems, osems))
    wid = lax.axis_index("s") * _NC + lax.axis_index("c")
    bbase = wid * _BPW
    pltpu.sync_copy(idx_hbm.at[wid], idx_v)

    def start_batch(i, rows, gsem):
        pltpu.async_copy(table_hbm.at[idx_v.at[2 * i]],
                         rows.at[pl.ds(0, _HALF)], gsem)
        pltpu.async_copy(table_hbm.at[idx_v.at[2 * i + 1]],
                         rows.at[pl.ds(_HALF, _HALF)], gsem)

    def wait_batch(i, rows, gsem):
        pltpu.make_async_copy(table_hbm.at[idx_v.at[2 * i]],
                              rows.at[pl.ds(0, _HALF)], gsem).wait()
        pltpu.make_async_copy(table_hbm.at[idx_v.at[2 * i + 1]],
                              rows.at[pl.ds(_HALF, _HALF)], gsem).wait()

    for b in range(_NBUF):
        start_batch(b, bufs[b][0], bufs[b][1])

    def body(g, _):
        for b in range(_NBUF):
            i = g * _NBUF + b
            rows, gsem, osem = bufs[b]
            wait_batch(i, rows, gsem)
            pltpu.async_copy(rows, out_hbm.at[bbase + i], osem)

            @pl.when(i + _NBUF < _BPW)
            def _():
                pltpu.make_async_copy(rows, out_hbm.at[bbase + i], osem).wait()
                start_batch(i + _NBUF, rows, gsem)

        return 0

    lax.fori_loop(0, _BPW // _NBUF, body, 0)

    for b in range(_NBUF):
        i = _BPW - _NBUF + b
        rows, _, osem = bufs[b]
        pltpu.make_async_copy(rows, out_hbm.at[bbase + i], osem).wait()


def kernel(inputs, embedding_table):
    idx = jnp.reshape(inputs.astype(jnp.int32), (_NW, 2 * _BPW, _HALF))
    out = _gather_kernel(idx, embedding_table)
    return (out, embedding_table)
